# Initial kernel scaffold; baseline (speedup 1.0000x reference)
import jax
import jax.numpy as jnp
from jax.experimental import pallas as pl


def kernel(edge_index, edge_weight):
    raise NotImplementedError("write your pallas kernel here")


# speedup vs baseline: 109.8381x; 109.8381x over previous
import functools

import numpy as np

import jax
import jax.numpy as jnp
from jax import lax
from jax.experimental import pallas as pl
from jax.experimental.pallas import tpu as pltpu
from jax.experimental.pallas import tpu_sc as plsc

N_NODES = 100000
N_EDGES = 6400000
NC = 2
NS = 16
L = 16
NW = NC * NS

SL = 6272
N_PAD = NS * SL
ECH = 10000
NCHG = N_EDGES // ECH
CPW = NCHG // NW

LI = np.int32(L)
SLI = np.int32(SL)
ECHI = np.int32(ECH)
CPWI = np.int32(CPW)
NCI = np.int32(NC)
I1 = np.int32(1)

_mesh = plsc.VectorSubcoreMesh(
    core_axis_name="c", subcore_axis_name="s", num_cores=NC, num_subcores=NS)
_params = pltpu.CompilerParams(needs_layout_passes=False)


def _sloop(length, body):
    def _step(i, _):
        body(i)
        return i + np.int32(1), None

    lax.scan(_step, np.int32(0), None, length=length)


@functools.partial(
    pl.kernel,
    out_type=jax.ShapeDtypeStruct((2, NC, NS, N_PAD), jnp.float32),
    mesh=_mesh,
    compiler_params=_params,
    scratch_types=[
        pltpu.VMEM((N_PAD,), jnp.float32),
        pltpu.VMEM((ECH,), jnp.int32),
        pltpu.VMEM((ECH,), jnp.float32),
    ],
)
def _degree_kernel(src_hbm, dst_hbm, w_hbm, out_hbm, tab, ixv, wv):
    c = lax.axis_index("c")
    s = lax.axis_index("s")
    wid = s * NCI + c
    t0 = wid * CPWI
    zero16 = jnp.zeros((L,), jnp.float32)

    for p, idx_hbm in ((np.int32(0), src_hbm), (np.int32(1), dst_hbm)):
        def _z(i):
            tab[pl.ds(i * LI, L)] = zero16

        _sloop(N_PAD // L, _z)

        def _chunk(i):
            e0 = (t0 + i) * ECHI
            pltpu.sync_copy(idx_hbm.at[pl.ds(e0, ECH)], ixv)
            pltpu.sync_copy(w_hbm.at[pl.ds(e0, ECH)], wv)

            def _acc(k):
                sl = pl.ds(k * LI, L)
                plsc.addupdate_scatter(tab, [ixv[sl]], wv[sl])

            _sloop(ECH // L, _acc)

        _sloop(CPW, _chunk)
        pltpu.sync_copy(tab, out_hbm.at[p, c, s])


@functools.partial(
    pl.kernel,
    out_type=jax.ShapeDtypeStruct((N_EDGES,), jnp.float32),
    mesh=_mesh,
    compiler_params=_params,
    scratch_types=[
        pltpu.VMEM_SHARED((N_PAD,), jnp.float32),
        pltpu.VMEM((N_PAD,), jnp.float32),
        pltpu.VMEM((ECH,), jnp.int32),
        pltpu.VMEM((ECH,), jnp.float32),
    ],
)
def _apply_kernel(src_hbm, dst_hbm, w_hbm, part_hbm, out_hbm,
                  norm_sh, tab, ixv, av):
    c = lax.axis_index("c")
    s = lax.axis_index("s")
    wid = s * NCI + c
    t0 = wid * CPWI

    for p, idx_hbm, val_hbm in ((np.int32(0), src_hbm, w_hbm),
                                (np.int32(1), dst_hbm, out_hbm)):
        acc = tab.at[pl.ds(0, SL)]
        tmp = tab.at[pl.ds(SL, SL)]
        first = True
        for cc in range(NC):
            for k in range(NS):
                dstbuf = acc if first else tmp
                pltpu.sync_copy(
                    part_hbm.at[p, np.int32(cc), np.int32(k),
                                pl.ds(s * SLI, SL)], dstbuf)
                if not first:
                    def _add(i):
                        sl = pl.ds(i * LI, L)
                        acc[sl] = acc[sl] + tmp[sl]

                    _sloop(SL // L, _add)
                first = False

        def _nrm(i):
            sl = pl.ds(i * LI, L)
            x = acc[sl]
            xi = plsc.bitcast(x, jnp.int32)
            yi = np.int32(0x5F3759DF) - (xi >> 1)
            y = plsc.bitcast(yi, jnp.float32)
            y = y * (1.5 - 0.5 * x * y * y)
            y = y * (1.5 - 0.5 * x * y * y)
            y = y * (1.5 - 0.5 * x * y * y)
            acc[sl] = y

        _sloop(SL // L, _nrm)

        pltpu.sync_copy(acc, norm_sh.at[pl.ds(s * SLI, SL)])
        plsc.subcore_barrier()
        pltpu.sync_copy(norm_sh, tab)
        plsc.subcore_barrier()

        def _chunk(i):
            e0 = (t0 + i) * ECHI
            pltpu.sync_copy(idx_hbm.at[pl.ds(e0, ECH)], ixv)
            pltpu.sync_copy(val_hbm.at[pl.ds(e0, ECH)], av)

            def _app(k):
                sl = pl.ds(k * LI, L)
                g = plsc.load_gather(tab, [ixv[sl]])
                av[sl] = av[sl] * g

            _sloop(ECH // L, _app)
            pltpu.sync_copy(av, out_hbm.at[pl.ds(e0, ECH)])

        _sloop(CPW, _chunk)


def kernel(edge_index, edge_weight):
    ei = edge_index.astype(jnp.int32)
    src = ei[0]
    dst = ei[1]
    w = edge_weight.astype(jnp.float32)
    parts = _degree_kernel(src, dst, w)
    return _apply_kernel(src, dst, w, parts)
